# Initial kernel scaffold; baseline (speedup 1.0000x reference)
import jax
import jax.numpy as jnp
from jax.experimental import pallas as pl


def kernel(inputs, kernel):
    raise NotImplementedError("write your pallas kernel here")


# speedup vs baseline: 14.2091x; 14.2091x over previous
import functools

import jax
import jax.numpy as jnp
from jax import lax
from jax.experimental import pallas as pl
from jax.experimental.pallas import tpu as pltpu
from jax.experimental.pallas import tpu_sc as plsc

_VOCAB = 100000
_D = 64
_NC, _NS = 2, 16
_NW = _NC * _NS
_CHUNK = 128


def _sc_gather(table2d, idx3d, n_chunks):
  mesh = plsc.VectorSubcoreMesh(core_axis_name="c", subcore_axis_name="s")

  @functools.partial(
      pl.kernel,
      out_type=jax.ShapeDtypeStruct((_NW, n_chunks, _CHUNK, _D), jnp.float32),
      mesh=mesh,
      scratch_types=[
          pltpu.VMEM((n_chunks, _CHUNK), jnp.int32),
          pltpu.VMEM((_CHUNK, _D), jnp.float32),
          pltpu.SemaphoreType.DMA,
      ],
      compiler_params=pltpu.CompilerParams(use_tc_tiling_on_sc=False),
  )
  def k(table_hbm, idx_hbm, out_hbm, idx_v, rows_v, sem):
    wid = lax.axis_index("s") * _NC + lax.axis_index("c")
    pltpu.sync_copy(idx_hbm.at[wid], idx_v)

    def body(j, carry):
      pltpu.async_copy(table_hbm.at[idx_v.at[j]], rows_v, sem).wait()
      pltpu.sync_copy(rows_v, out_hbm.at[wid, j])
      return carry

    lax.fori_loop(0, n_chunks, body, 0)

  return k(table2d, idx3d)


def kernel(inputs, kernel):
  b, s = inputs.shape
  total = b * s
  n_chunks = total // (_NW * _CHUNK)
  table2d = kernel.reshape(_VOCAB, _D)
  idx = inputs.reshape(_NW, n_chunks, _CHUNK).astype(jnp.int32)
  out = _sc_gather(table2d, idx, n_chunks)
  return out.reshape(b, s, 8, 8)


# speedup vs baseline: 15.0787x; 1.0612x over previous
import functools

import jax
import jax.numpy as jnp
from jax import lax
from jax.experimental import pallas as pl
from jax.experimental.pallas import tpu as pltpu
from jax.experimental.pallas import tpu_sc as plsc

_VOCAB = 100000
_D = 64
_NC, _NS = 2, 16
_NW = _NC * _NS
_CHUNK = 640


def _sc_gather(table2d, idx3d, n_chunks):
  mesh = plsc.VectorSubcoreMesh(core_axis_name="c", subcore_axis_name="s")

  @functools.partial(
      pl.kernel,
      out_type=jax.ShapeDtypeStruct((_NW, n_chunks, _CHUNK, _D), jnp.float32),
      mesh=mesh,
      scratch_types=[
          pltpu.VMEM((n_chunks, _CHUNK), jnp.int32),
          pltpu.VMEM((_CHUNK, _D), jnp.float32),
          pltpu.SemaphoreType.DMA,
      ],
      compiler_params=pltpu.CompilerParams(use_tc_tiling_on_sc=False),
  )
  def k(table_hbm, idx_hbm, out_hbm, idx_v, rows_v, sem):
    wid = lax.axis_index("s") * _NC + lax.axis_index("c")
    pltpu.sync_copy(idx_hbm.at[wid], idx_v)

    def body(j, carry):
      pltpu.async_copy(table_hbm.at[idx_v.at[j]], rows_v, sem).wait()
      pltpu.sync_copy(rows_v, out_hbm.at[wid, j])
      return carry

    lax.fori_loop(0, n_chunks, body, 0)

  return k(table2d, idx3d)


def kernel(inputs, kernel):
  b, s = inputs.shape
  total = b * s
  n_chunks = total // (_NW * _CHUNK)
  table2d = kernel.reshape(_VOCAB, _D)
  idx = inputs.reshape(_NW, n_chunks, _CHUNK).astype(jnp.int32)
  out = _sc_gather(table2d, idx, n_chunks)
  return out.reshape(b, s, 8, 8)
